# baseline (speedup 1.0000x reference)
import functools

import jax
import jax.numpy as jnp
from jax.experimental import pallas as pl
from jax.experimental.pallas import tpu as pltpu

R = 256
C = 256
VMEM_LIMIT = 60 * 1024 * 1024


def _spmm_layer_kernel(nchunks_ref, cbase_ref, z_ref, src_ref, pw_ref,
                       dl_ref, w_ref, b_ref, s_ref, zs_ref, out_ref,
                       mtile, acc, idx_smem, sems, *, log_softmax):
    i = pl.program_id(0)
    nc = nchunks_ref[i]
    cb = cbase_ref[i]

    acc[...] = jnp.zeros_like(acc)

    @pl.when(nc > 0)
    def _():
        pltpu.make_async_copy(src_ref.at[cb, 0], idx_smem.at[0],
                              sems.at[0]).start()

    def chunk_body(j, _):
        slot = jax.lax.rem(j, 2)
        nslot = 1 - slot

        @pl.when(j + 1 < nc)
        def _():
            pltpu.make_async_copy(src_ref.at[cb + j + 1, 0],
                                  idx_smem.at[nslot], sems.at[nslot]).start()

        pltpu.make_async_copy(src_ref.at[cb + j, 0], idx_smem.at[slot],
                              sems.at[slot]).wait()

        for c in range(C):
            idx = idx_smem[slot, c]
            base = pl.multiple_of((idx >> 3) << 3, 8)
            chunk8 = z_ref[pl.ds(base, 8), :]
            row = pltpu.roll(chunk8, -(idx & 7), axis=0)[0:1, :]
            mtile[pl.ds(c, 1), :] = row

        pw = pw_ref[cb + j]
        dl = dl_ref[cb + j]
        riota = jax.lax.broadcasted_iota(jnp.int32, (R, C), 0)
        p = jnp.where(riota == dl, pw, 0.0)
        acc[...] += jnp.dot(p, mtile[...],
                            preferred_element_type=jnp.float32)
        return 0

    jax.lax.fori_loop(0, nc, chunk_body, 0)

    h_pre = acc[...] + s_ref[...] * zs_ref[...]
    h = jnp.dot(h_pre, w_ref[...], preferred_element_type=jnp.float32)
    h = jnp.maximum(h + b_ref[...], 0.0)
    if log_softmax:
        m = jnp.max(h, axis=1, keepdims=True)
        zc = h - m
        lse = jnp.log(jnp.sum(jnp.exp(zc), axis=1, keepdims=True))
        h = zc - lse
    out_ref[...] = h


def _spmm_layer(nchunks, cbase, z, srcs, pws, dls, w, b, s,
                *, log_softmax):
    n, f = z.shape
    d_out = w.shape[1]
    nb = n // R
    nchunks_tot = srcs.shape[0]

    kernel_body = functools.partial(_spmm_layer_kernel,
                                    log_softmax=log_softmax)
    e_pad = nchunks_tot * C
    cost = pl.CostEstimate(
        flops=int(2 * e_pad * R * f + 2 * n * f * d_out),
        transcendentals=int(n * d_out if log_softmax else 0),
        bytes_accessed=int(n * f * 4 * 2 + 3 * e_pad * 4 + n * d_out * 4),
    )

    return pl.pallas_call(
        kernel_body,
        out_shape=jax.ShapeDtypeStruct((n, d_out), jnp.float32),
        grid_spec=pltpu.PrefetchScalarGridSpec(
            num_scalar_prefetch=2,
            grid=(nb,),
            in_specs=[
                pl.BlockSpec((n, f), lambda i, *_: (0, 0)),
                pl.BlockSpec((nchunks_tot, 1, C), lambda i, *_: (0, 0, 0)),
                pl.BlockSpec((nchunks_tot, 1, C), lambda i, *_: (0, 0, 0)),
                pl.BlockSpec((nchunks_tot, 1, C), lambda i, *_: (0, 0, 0)),
                pl.BlockSpec((f, d_out), lambda i, *_: (0, 0)),
                pl.BlockSpec((1, d_out), lambda i, *_: (0, 0)),
                pl.BlockSpec((R, 1), lambda i, *_: (i, 0)),
                pl.BlockSpec((R, f), lambda i, *_: (i, 0)),
            ],
            out_specs=pl.BlockSpec((R, d_out), lambda i, *_: (i, 0)),
            scratch_shapes=[
                pltpu.VMEM((C, 128), jnp.float32),
                pltpu.VMEM((R, 128), jnp.float32),
                pltpu.SMEM((2, C), jnp.int32),
                pltpu.SemaphoreType.DMA((2,)),
            ],
        ),
        compiler_params=pltpu.CompilerParams(
            dimension_semantics=("parallel",),
            vmem_limit_bytes=VMEM_LIMIT),
        cost_estimate=cost,
    )(nchunks, cbase, z, srcs, pws, dls, w, b, s, z)


def _prep_edges(edge_index, n):
    src, dst = edge_index[0], edge_index[1]
    e = src.shape[0]
    nb = n // R

    ones = jnp.ones((e,), jnp.float32)
    deg = jnp.zeros((n,), jnp.float32).at[dst].add(ones)
    selfcnt = jnp.zeros((n,), jnp.float32).at[dst].add(
        jnp.where(src == dst, 1.0, 0.0))
    no_self = selfcnt == 0.0
    deg = deg + jnp.where(no_self, 1.0, 0.0)
    dis = jax.lax.rsqrt(jnp.maximum(deg, 1.0))
    wgt = dis[dst] * dis[src]
    s_vec = jnp.where(no_self, dis * dis, 0.0).astype(jnp.float32)

    order = jnp.argsort(dst)
    src_s = src[order]
    dst_s = dst[order]
    w_s = wgt[order]
    blk = dst_s // R

    cnt = jnp.zeros((nb,), jnp.int32).at[blk].add(1)
    off = jnp.concatenate([jnp.zeros((1,), jnp.int32),
                           jnp.cumsum(cnt)[:-1]])
    pcnt = ((cnt + C - 1) // C) * C
    poff = jnp.concatenate([jnp.zeros((1,), jnp.int32),
                            jnp.cumsum(pcnt)[:-1]])
    pos = poff[blk] + (jnp.arange(e, dtype=jnp.int32) - off[blk])

    e_pad = e + nb * C
    srcs = jnp.zeros((e_pad,), jnp.int32).at[pos].set(src_s)
    pws = jnp.zeros((e_pad,), jnp.float32).at[pos].set(w_s)
    dls = jnp.zeros((e_pad,), jnp.int32).at[pos].set(
        (dst_s % R).astype(jnp.int32))

    nchunks = pcnt // C
    cbase = poff // C
    nch_tot = e_pad // C
    return (srcs.reshape(nch_tot, 1, C), pws.reshape(nch_tot, 1, C),
            dls.reshape(nch_tot, 1, C), nchunks, cbase,
            s_vec.reshape(n, 1))


def kernel(x_ids, edge_index, edge_attr, embed_weight, w1, b1, w2, b2):
    del edge_attr
    x = embed_weight[x_ids].astype(jnp.float32)
    n = x.shape[0]

    srcs, pws, dls, nchunks, cbase, s_vec = _prep_edges(edge_index, n)

    w1f = w1.astype(jnp.float32)
    w2f = w2.astype(jnp.float32)
    b1f = b1.astype(jnp.float32)
    b2f = b2.astype(jnp.float32)

    h1 = _spmm_layer(nchunks, cbase, x, srcs, pws, dls, w1f, b1f, s_vec,
                     log_softmax=False)
    out = _spmm_layer(nchunks, cbase, h1, srcs, pws, dls, w2f, b2f, s_vec,
                      log_softmax=True)
    return out


# speedup vs baseline: 1.2916x; 1.2916x over previous
import functools

import jax
import jax.numpy as jnp
from jax.experimental import pallas as pl
from jax.experimental.pallas import tpu as pltpu

R = 256
C = 256
VMEM_LIMIT = 60 * 1024 * 1024


def _spmm_layer_kernel(nchunks_ref, cbase_ref, z_ref, src_ref, pw_ref,
                       dl_ref, w_ref, b_ref, s_ref, zs_ref, out_ref,
                       mtile, acc, idx_smem, sems, *, log_softmax):
    i = pl.program_id(0)
    nc = nchunks_ref[i]
    cb = cbase_ref[i]

    acc[...] = jnp.zeros_like(acc)

    @pl.when(nc > 0)
    def _():
        pltpu.make_async_copy(src_ref.at[cb, 0], idx_smem.at[0],
                              sems.at[0]).start()

    def chunk_body(j, _):
        slot = jax.lax.rem(j, 2)
        nslot = 1 - slot

        @pl.when(j + 1 < nc)
        def _():
            pltpu.make_async_copy(src_ref.at[cb + j + 1, 0],
                                  idx_smem.at[nslot], sems.at[nslot]).start()

        pltpu.make_async_copy(src_ref.at[cb + j, 0], idx_smem.at[slot],
                              sems.at[slot]).wait()

        for c in range(C):
            idx = idx_smem[slot, c]
            base = pl.multiple_of((idx >> 3) << 3, 8)
            chunk8 = z_ref[pl.ds(base, 8), :]
            row = pltpu.roll(chunk8, -(idx & 7), axis=0)[0:1, :]
            mtile[pl.ds(c, 1), :] = row

        pw = pw_ref[cb + j]
        dl = dl_ref[cb + j]
        riota = jax.lax.broadcasted_iota(jnp.int32, (R, C), 0)
        p = jnp.where(riota == dl, pw, 0.0)
        acc[...] += jnp.dot(p, mtile[...],
                            preferred_element_type=jnp.float32)
        return 0

    jax.lax.fori_loop(0, nc, chunk_body, 0)

    h_pre = acc[...] + s_ref[...] * zs_ref[...]
    h = jnp.dot(h_pre, w_ref[...], preferred_element_type=jnp.float32)
    h = jnp.maximum(h + b_ref[...], 0.0)
    if log_softmax:
        m = jnp.max(h, axis=1, keepdims=True)
        zc = h - m
        lse = jnp.log(jnp.sum(jnp.exp(zc), axis=1, keepdims=True))
        h = zc - lse
    out_ref[...] = h


def _spmm_layer(nchunks, cbase, z, srcs, pws, dls, w, b, s,
                *, log_softmax):
    n, f = z.shape
    d_out = w.shape[1]
    nb = n // R
    nchunks_tot = srcs.shape[0]

    kernel_body = functools.partial(_spmm_layer_kernel,
                                    log_softmax=log_softmax)
    e_pad = nchunks_tot * C
    cost = pl.CostEstimate(
        flops=int(2 * e_pad * R * f + 2 * n * f * d_out),
        transcendentals=int(n * d_out if log_softmax else 0),
        bytes_accessed=int(n * f * 4 * 2 + 3 * e_pad * 4 + n * d_out * 4),
    )

    return pl.pallas_call(
        kernel_body,
        out_shape=jax.ShapeDtypeStruct((n, d_out), jnp.float32),
        grid_spec=pltpu.PrefetchScalarGridSpec(
            num_scalar_prefetch=2,
            grid=(nb,),
            in_specs=[
                pl.BlockSpec((n, f), lambda i, *_: (0, 0)),
                pl.BlockSpec((nchunks_tot, 1, C), lambda i, *_: (0, 0, 0)),
                pl.BlockSpec((nchunks_tot, 1, C), lambda i, *_: (0, 0, 0)),
                pl.BlockSpec((nchunks_tot, 1, C), lambda i, *_: (0, 0, 0)),
                pl.BlockSpec((f, d_out), lambda i, *_: (0, 0)),
                pl.BlockSpec((1, d_out), lambda i, *_: (0, 0)),
                pl.BlockSpec((R, 1), lambda i, *_: (i, 0)),
                pl.BlockSpec((R, f), lambda i, *_: (i, 0)),
            ],
            out_specs=pl.BlockSpec((R, d_out), lambda i, *_: (i, 0)),
            scratch_shapes=[
                pltpu.VMEM((C, 128), jnp.float32),
                pltpu.VMEM((R, 128), jnp.float32),
                pltpu.SMEM((2, C), jnp.int32),
                pltpu.SemaphoreType.DMA((2,)),
            ],
        ),
        compiler_params=pltpu.CompilerParams(
            dimension_semantics=("parallel",),
            vmem_limit_bytes=VMEM_LIMIT),
        cost_estimate=cost,
    )(nchunks, cbase, z, srcs, pws, dls, w, b, s, z)


def _prep_edges(edge_index, n):
    src, dst = edge_index[0], edge_index[1]
    e = src.shape[0]
    nb = n // R

    ones = jnp.ones((e,), jnp.float32)
    deg = jnp.zeros((n,), jnp.float32).at[dst].add(ones)
    selfcnt = jnp.zeros((n,), jnp.float32).at[dst].add(
        jnp.where(src == dst, 1.0, 0.0))
    no_self = selfcnt == 0.0
    deg = deg + jnp.where(no_self, 1.0, 0.0)
    dis = jax.lax.rsqrt(jnp.maximum(deg, 1.0))
    wgt = dis[dst] * dis[src]
    s_vec = jnp.where(no_self, dis * dis, 0.0).astype(jnp.float32)

    blk = (dst // R).astype(jnp.int32)
    g = 128
    e_grp = ((e + g - 1) // g) * g
    n_grp = e_grp // g
    blkg = jnp.full((e_grp,), nb, jnp.int32).at[:e].set(blk).reshape(n_grp, g)

    tril = (jax.lax.broadcasted_iota(jnp.int32, (g, g), 1)
            < jax.lax.broadcasted_iota(jnp.int32, (g, g), 0))
    same = blkg[:, :, None] == blkg[:, None, :]
    r1 = jnp.sum(jnp.where(same & tril[None], 1, 0), axis=2,
                 dtype=jnp.int32).reshape(e_grp)[:e]
    biota = jax.lax.broadcasted_iota(jnp.int32, (n_grp, g, nb), 2)
    cnt2 = jnp.sum(jnp.where(blkg[:, :, None] == biota, 1, 0), axis=1,
                   dtype=jnp.int32)
    base2 = jnp.cumsum(cnt2, axis=0) - cnt2
    cnt = base2[-1] + cnt2[-1]

    pcnt = ((cnt + C - 1) // C) * C
    poff = jnp.concatenate([jnp.zeros((1,), jnp.int32),
                            jnp.cumsum(pcnt)[:-1]])
    g_idx = jnp.arange(e, dtype=jnp.int32) // g
    rank_in_blk = base2.reshape(-1)[g_idx * nb + blk] + r1
    pos = poff[blk] + rank_in_blk

    e_pad = e + nb * C
    srcs = jnp.zeros((e_pad,), jnp.int32).at[pos].set(src)
    pws = jnp.zeros((e_pad,), jnp.float32).at[pos].set(wgt)
    dls = jnp.zeros((e_pad,), jnp.int32).at[pos].set(
        (dst % R).astype(jnp.int32))

    nchunks = pcnt // C
    cbase = poff // C
    nch_tot = e_pad // C
    return (srcs.reshape(nch_tot, 1, C), pws.reshape(nch_tot, 1, C),
            dls.reshape(nch_tot, 1, C), nchunks, cbase,
            s_vec.reshape(n, 1))


def kernel(x_ids, edge_index, edge_attr, embed_weight, w1, b1, w2, b2):
    del edge_attr
    x = embed_weight[x_ids].astype(jnp.float32)
    n = x.shape[0]

    srcs, pws, dls, nchunks, cbase, s_vec = _prep_edges(edge_index, n)

    w1f = w1.astype(jnp.float32)
    w2f = w2.astype(jnp.float32)
    b1f = b1.astype(jnp.float32)
    b2f = b2.astype(jnp.float32)

    h1 = _spmm_layer(nchunks, cbase, x, srcs, pws, dls, w1f, b1f, s_vec,
                     log_softmax=False)
    out = _spmm_layer(nchunks, cbase, h1, srcs, pws, dls, w2f, b2f, s_vec,
                      log_softmax=True)
    return out
